# speedup vs baseline: 6.4508x; 6.4508x over previous
import jax
import jax.numpy as jnp
from jax import lax
from jax.experimental import pallas as pl
from jax.experimental.pallas import tpu as pltpu
from jax.experimental.pallas import tpu_sc as plsc

N_OUT = 10242
N_IN = 40962
FAN = 7
D = 512
LANES = 16
NW = 32
C = 8
PW = 336
K = PW // C
PAD_N = NW * PW
IDX_PW = PW * FAN

_INV_FAN = 1.0 / FAN


def _pool_kernel(x_hbm, idx_hbm, out_hbm, idx_v, gbuf, obuf, gsem, osem):
    cid = lax.axis_index("c")
    sid = lax.axis_index("s")
    wid = sid * 2 + cid
    row0 = wid * PW

    pltpu.sync_copy(idx_hbm.at[pl.ds(wid * IDX_PW, IDX_PW)], idx_v)

    def gather_start(g, buf_slot):
        idx_slice = idx_v.at[pl.ds(g * (C * FAN), C * FAN)]
        pltpu.async_copy(x_hbm.at[idx_slice], gbuf.at[buf_slot], gsem.at[buf_slot])

    def gather_wait(buf_slot):
        pltpu.make_async_copy(
            x_hbm.at[idx_v.at[pl.ds(0, C * FAN)]], gbuf.at[buf_slot],
            gsem.at[buf_slot],
        ).wait()

    def out_start(g, buf_slot):
        base = row0 + g * C
        nval = N_OUT - base

        @pl.when(nval >= C)
        def _full():
            pltpu.async_copy(
                obuf.at[buf_slot], out_hbm.at[pl.ds(base, C), :], osem.at[buf_slot]
            )

        @pl.when(jnp.logical_and(nval > 0, nval < C))
        def _part():
            pltpu.async_copy(
                obuf.at[buf_slot, pl.ds(0, 2), :],
                out_hbm.at[pl.ds(base, 2), :],
                osem.at[buf_slot],
            )

    def out_wait(g, buf_slot):
        base = row0 + g * C
        nval = N_OUT - base

        @pl.when(nval >= C)
        def _full():
            pltpu.make_async_copy(
                obuf.at[buf_slot], out_hbm.at[pl.ds(0, C), :], osem.at[buf_slot]
            ).wait()

        @pl.when(jnp.logical_and(nval > 0, nval < C))
        def _part():
            pltpu.make_async_copy(
                obuf.at[buf_slot, pl.ds(0, 2), :],
                out_hbm.at[pl.ds(0, 2), :],
                osem.at[buf_slot],
            ).wait()

    def compute_chunk(buf_slot):
        @pl.loop(0, C)
        def _rows(r):
            rbase = r * FAN
            for j in range(D // LANES):
                sl = pl.ds(j * LANES, LANES)
                acc = gbuf[buf_slot, rbase, sl]
                for k in range(1, FAN):
                    acc = acc + gbuf[buf_slot, rbase + k, sl]
                obuf[buf_slot, r, sl] = acc * _INV_FAN

    gather_start(0, 0)

    @pl.loop(0, K, step=2)
    def _chunks(g):
        for b in range(2):
            gg = g + b

            @pl.when(gg + 1 < K)
            def _next():
                gather_start(gg + 1, (b + 1) % 2)

            gather_wait(b)

            @pl.when(gg >= 2)
            def _drain():
                out_wait(gg - 2, b)

            compute_chunk(b)
            out_start(gg, b)

    out_wait(K - 2, 0)
    out_wait(K - 1, 1)


@jax.jit
def kernel(x, edge_index):
    src = edge_index[1].astype(jnp.int32)
    idx = src.reshape(N_OUT, FAN)
    idx = jnp.concatenate(
        [idx, jnp.zeros((PAD_N - N_OUT, FAN), jnp.int32)], axis=0
    ).reshape(-1)

    mesh = plsc.VectorSubcoreMesh(core_axis_name="c", subcore_axis_name="s")
    run = pl.kernel(
        _pool_kernel,
        out_type=jax.ShapeDtypeStruct((N_OUT, D), jnp.float32),
        mesh=mesh,
        scratch_types=[
            pltpu.VMEM((IDX_PW,), jnp.int32),
            pltpu.VMEM((2, C * FAN, D), jnp.float32),
            pltpu.VMEM((2, C, D), jnp.float32),
            pltpu.SemaphoreType.DMA((2,)),
            pltpu.SemaphoreType.DMA((2,)),
        ],
    )
    return run(x, idx)


# speedup vs baseline: 6.4750x; 1.0037x over previous
import jax
import jax.numpy as jnp
from jax import lax
from jax.experimental import pallas as pl
from jax.experimental.pallas import tpu as pltpu
from jax.experimental.pallas import tpu_sc as plsc

N_OUT = 10242
N_IN = 40962
FAN = 7
D = 512
LANES = 16
NW = 32
C = 8
PW = 336
K = PW // C
PAD_N = NW * PW
IDX_PW = PW * FAN

_INV_FAN = 1.0 / FAN


def _pool_kernel(x_hbm, idx_hbm, out_hbm, idx_v, gbuf, obuf, gsem, osem):
    cid = lax.axis_index("c")
    sid = lax.axis_index("s")
    wid = sid * 2 + cid
    row0 = wid * PW

    pltpu.sync_copy(idx_hbm.at[pl.ds(wid * IDX_PW, IDX_PW)], idx_v)

    def gather_start(g, buf_slot):
        idx_slice = idx_v.at[pl.ds(g * (C * FAN), C * FAN)]
        pltpu.async_copy(x_hbm.at[idx_slice], gbuf.at[buf_slot], gsem.at[buf_slot])

    def gather_wait(buf_slot):
        pltpu.make_async_copy(
            x_hbm.at[idx_v.at[pl.ds(0, C * FAN)]], gbuf.at[buf_slot],
            gsem.at[buf_slot],
        ).wait()

    def out_start(g, buf_slot):
        base = row0 + g * C
        nval = N_OUT - base

        @pl.when(nval >= C)
        def _full():
            pltpu.async_copy(
                obuf.at[buf_slot], out_hbm.at[pl.ds(base, C), :], osem.at[buf_slot]
            )

        @pl.when(jnp.logical_and(nval > 0, nval < C))
        def _part():
            pltpu.async_copy(
                obuf.at[buf_slot, pl.ds(0, 2), :],
                out_hbm.at[pl.ds(base, 2), :],
                osem.at[buf_slot],
            )

    def out_wait(g, buf_slot):
        base = row0 + g * C
        nval = N_OUT - base

        @pl.when(nval >= C)
        def _full():
            pltpu.make_async_copy(
                obuf.at[buf_slot], out_hbm.at[pl.ds(0, C), :], osem.at[buf_slot]
            ).wait()

        @pl.when(jnp.logical_and(nval > 0, nval < C))
        def _part():
            pltpu.make_async_copy(
                obuf.at[buf_slot, pl.ds(0, 2), :],
                out_hbm.at[pl.ds(0, 2), :],
                osem.at[buf_slot],
            ).wait()

    def compute_chunk(buf_slot):
        @pl.loop(0, C)
        def _rows(r):
            rbase = r * FAN

            def load_group(j):
                sl = pl.ds(j * LANES, LANES)
                return [gbuf[buf_slot, rbase + k, sl] for k in range(FAN)]

            def reduce_store(j, v):
                acc = ((v[0] + v[1]) + (v[2] + v[3])) + ((v[4] + v[5]) + v[6])
                obuf[buf_slot, r, pl.ds(j * LANES, LANES)] = acc * _INV_FAN

            prev = load_group(0)
            for j in range(1, D // LANES):
                cur = load_group(j)
                reduce_store(j - 1, prev)
                prev = cur
            reduce_store(D // LANES - 1, prev)

    gather_start(0, 0)

    @pl.loop(0, K, step=2)
    def _chunks(g):
        for b in range(2):
            gg = g + b

            @pl.when(gg + 1 < K)
            def _next():
                gather_start(gg + 1, (b + 1) % 2)

            gather_wait(b)

            @pl.when(gg >= 2)
            def _drain():
                out_wait(gg - 2, b)

            compute_chunk(b)
            out_start(gg, b)

    out_wait(K - 2, 0)
    out_wait(K - 1, 1)


@jax.jit
def kernel(x, edge_index):
    src = edge_index[1].astype(jnp.int32)
    idx = src.reshape(N_OUT, FAN)
    idx = jnp.concatenate(
        [idx, jnp.zeros((PAD_N - N_OUT, FAN), jnp.int32)], axis=0
    ).reshape(-1)

    mesh = plsc.VectorSubcoreMesh(core_axis_name="c", subcore_axis_name="s")
    run = pl.kernel(
        _pool_kernel,
        out_type=jax.ShapeDtypeStruct((N_OUT, D), jnp.float32),
        mesh=mesh,
        scratch_types=[
            pltpu.VMEM((IDX_PW,), jnp.int32),
            pltpu.VMEM((2, C * FAN, D), jnp.float32),
            pltpu.VMEM((2, C, D), jnp.float32),
            pltpu.SemaphoreType.DMA((2,)),
            pltpu.SemaphoreType.DMA((2,)),
        ],
    )
    return run(x, idx)


# speedup vs baseline: 6.5033x; 1.0044x over previous
import jax
import jax.numpy as jnp
from jax import lax
from jax.experimental import pallas as pl
from jax.experimental.pallas import tpu as pltpu
from jax.experimental.pallas import tpu_sc as plsc

N_OUT = 10242
N_IN = 40962
FAN = 7
D = 512
LANES = 16
NW = 32
C = 8
PW = 336
K = PW // C
PAD_N = NW * PW
IDX_PW = PW * FAN

_INV_FAN = 1.0 / FAN


def _pool_kernel(x_hbm, idx_hbm, out_hbm, idx_v, gbuf, obuf, gsem, osem):
    cid = lax.axis_index("c")
    sid = lax.axis_index("s")
    wid = sid * 2 + cid
    row0 = wid * PW

    pltpu.sync_copy(idx_hbm.at[pl.ds(wid * IDX_PW, IDX_PW)], idx_v)

    def gather_start(g, buf_slot):
        idx_slice = idx_v.at[pl.ds(g * (C * FAN), C * FAN)]
        pltpu.async_copy(x_hbm.at[idx_slice], gbuf.at[buf_slot], gsem.at[buf_slot])

    def gather_wait(buf_slot):
        pltpu.make_async_copy(
            x_hbm.at[idx_v.at[pl.ds(0, C * FAN)]], gbuf.at[buf_slot],
            gsem.at[buf_slot],
        ).wait()

    def out_start(g, buf_slot):
        base = row0 + g * C
        nval = N_OUT - base

        @pl.when(nval >= C)
        def _full():
            pltpu.async_copy(
                obuf.at[buf_slot], out_hbm.at[pl.ds(base, C), :], osem.at[buf_slot]
            )

        @pl.when(jnp.logical_and(nval > 0, nval < C))
        def _part():
            pltpu.async_copy(
                obuf.at[buf_slot, pl.ds(0, 2), :],
                out_hbm.at[pl.ds(base, 2), :],
                osem.at[buf_slot],
            )

    def out_wait(g, buf_slot):
        base = row0 + g * C
        nval = N_OUT - base

        @pl.when(nval >= C)
        def _full():
            pltpu.make_async_copy(
                obuf.at[buf_slot], out_hbm.at[pl.ds(0, C), :], osem.at[buf_slot]
            ).wait()

        @pl.when(jnp.logical_and(nval > 0, nval < C))
        def _part():
            pltpu.make_async_copy(
                obuf.at[buf_slot, pl.ds(0, 2), :],
                out_hbm.at[pl.ds(0, 2), :],
                osem.at[buf_slot],
            ).wait()

    def compute_chunk(buf_slot):
        @pl.loop(0, C)
        def _rows(r):
            rbase = r * FAN

            def load_group(j):
                sl = pl.ds(j * LANES, LANES)
                return [gbuf[buf_slot, rbase + k, sl] for k in range(FAN)]

            def reduce_store(j, v):
                acc = ((v[0] + v[1]) + (v[2] + v[3])) + ((v[4] + v[5]) + v[6])
                obuf[buf_slot, r, pl.ds(j * LANES, LANES)] = acc * _INV_FAN

            prev = load_group(0)
            for j in range(1, D // LANES):
                cur = load_group(j)
                reduce_store(j - 1, prev)
                prev = cur
            reduce_store(D // LANES - 1, prev)

    gather_start(0, 0)
    gather_start(1, 1)

    @pl.loop(0, K, step=3)
    def _chunks(g):
        for b in range(3):
            gg = g + b

            @pl.when(gg + 2 < K)
            def _next():
                gather_start(gg + 2, (b + 2) % 3)

            gather_wait(b)

            @pl.when(gg >= 3)
            def _drain():
                out_wait(gg - 3, b)

            compute_chunk(b)
            out_start(gg, b)

    out_wait(K - 3, 0)
    out_wait(K - 2, 1)
    out_wait(K - 1, 2)


@jax.jit
def kernel(x, edge_index):
    src = edge_index[1].astype(jnp.int32)
    idx = src.reshape(N_OUT, FAN)
    idx = jnp.concatenate(
        [idx, jnp.zeros((PAD_N - N_OUT, FAN), jnp.int32)], axis=0
    ).reshape(-1)

    mesh = plsc.VectorSubcoreMesh(core_axis_name="c", subcore_axis_name="s")
    run = pl.kernel(
        _pool_kernel,
        out_type=jax.ShapeDtypeStruct((N_OUT, D), jnp.float32),
        mesh=mesh,
        scratch_types=[
            pltpu.VMEM((IDX_PW,), jnp.int32),
            pltpu.VMEM((3, C * FAN, D), jnp.float32),
            pltpu.VMEM((3, C, D), jnp.float32),
            pltpu.SemaphoreType.DMA((3,)),
            pltpu.SemaphoreType.DMA((3,)),
        ],
    )
    return run(x, idx)
